# baseline (speedup 1.0000x reference)
import functools
import jax
import jax.numpy as jnp
from jax import lax
from jax.experimental import pallas as pl
from jax.experimental.pallas import tpu as pltpu
from jax.experimental.pallas import tpu_sc as plsc

TIME_STEP = 0.001
T = 1000
B = 256
C = 256
NC = 2
NS = 16
L = 16
NW = NC * NS
ROWS_PER_W = B // NW
T_CHUNK = 200
N_CHUNK = T // T_CHUNK
CHUNKS = ROWS_PER_W * N_CHUNK


@functools.partial(
    pl.kernel,
    out_type=jax.ShapeDtypeStruct((B, T, C), jnp.float32),
    mesh=plsc.VectorSubcoreMesh(core_axis_name="c", subcore_axis_name="s"),
    scratch_types=[
        pltpu.VMEM((ROWS_PER_W, C), jnp.float32),
        pltpu.VMEM((T_CHUNK, C), jnp.float32),
        pltpu.VMEM((T_CHUNK, C), jnp.float32),
        pltpu.SemaphoreType.DMA,
        pltpu.SemaphoreType.DMA,
    ],
    compiler_params=pltpu.CompilerParams(needs_layout_passes=False),
)
def _sc_one_hot(x_hbm, out_hbm, xrows_v, buf_a, buf_b, sem_a, sem_b):
    wid = lax.axis_index("s") * NC + lax.axis_index("c")
    row0 = wid * ROWS_PER_W
    pltpu.sync_copy(x_hbm.at[pl.ds(row0, ROWS_PER_W)], xrows_v)

    ones16 = jnp.ones((L,), jnp.float32)
    zeros16 = jnp.zeros((L,), jnp.float32)
    col_iota = lax.iota(jnp.int32, L)

    def zero(buf):
        def body(i, carry):
            for j in range(C // L):
                buf[i, pl.ds(j * L, L)] = jnp.zeros((L,), jnp.float32)
            return carry

        lax.fori_loop(0, T_CHUNK, body, 0)

    def put(c, buf, val):
        r = c // N_CHUNK
        t0 = (c % N_CHUNK) * T_CHUNK
        for j in range(C // L):
            xv = xrows_v[r, pl.ds(j * L, L)]
            rr = (xv / TIME_STEP).astype(jnp.int32) - t0
            m = (rr >= 0) & (rr < T_CHUNK)
            cols = col_iota + (j * L)
            plsc.store_scatter(buf, [rr, cols], val, mask=m)

    def copy(c, buf, sem):
        r = c // N_CHUNK
        t0 = pl.multiple_of((c % N_CHUNK) * T_CHUNK, T_CHUNK)
        return pltpu.make_async_copy(
            buf, out_hbm.at[row0 + r, pl.ds(t0, T_CHUNK)], sem
        )

    zero(buf_a)
    put(0, buf_a, ones16)
    copy(0, buf_a, sem_a).start()
    zero(buf_b)
    put(1, buf_b, ones16)
    copy(1, buf_b, sem_b).start()

    def pair(p, carry):
        c0 = 2 * p
        copy(c0 - 2, buf_a, sem_a).wait()
        put(c0 - 2, buf_a, zeros16)
        put(c0, buf_a, ones16)
        copy(c0, buf_a, sem_a).start()
        copy(c0 - 1, buf_b, sem_b).wait()
        put(c0 - 1, buf_b, zeros16)
        put(c0 + 1, buf_b, ones16)
        copy(c0 + 1, buf_b, sem_b).start()
        return carry

    lax.fori_loop(1, CHUNKS // 2, pair, 0)

    copy(CHUNKS - 2, buf_a, sem_a).wait()
    copy(CHUNKS - 1, buf_b, sem_b).wait()


def kernel(x):
    return _sc_one_hot(x)


# speedup vs baseline: 1.0078x; 1.0078x over previous
import functools
import jax
import jax.numpy as jnp
from jax import lax
from jax.experimental import pallas as pl
from jax.experimental.pallas import tpu as pltpu
from jax.experimental.pallas import tpu_sc as plsc

TIME_STEP = 0.001
T = 1000
B = 256
C = 256
NC = 2
NS = 16
L = 16
NW = NC * NS
ROWS_PER_W = B // NW
T_CHUNK = 200
N_CHUNK = T // T_CHUNK
CHUNKS = ROWS_PER_W * N_CHUNK


@functools.partial(
    pl.kernel,
    out_type=jax.ShapeDtypeStruct((B, T, C), jnp.float32),
    mesh=plsc.VectorSubcoreMesh(core_axis_name="c", subcore_axis_name="s"),
    scratch_types=[
        pltpu.VMEM((ROWS_PER_W, C), jnp.float32),
        pltpu.VMEM((T_CHUNK, C), jnp.float32),
        pltpu.VMEM((T_CHUNK, C), jnp.float32),
        pltpu.SemaphoreType.DMA,
        pltpu.SemaphoreType.DMA,
    ],
    compiler_params=pltpu.CompilerParams(needs_layout_passes=False),
)
def _sc_one_hot(x_hbm, out_hbm, xrows_v, buf_a, buf_b, sem_a, sem_b):
    wid = lax.axis_index("s") * NC + lax.axis_index("c")
    row0 = wid * ROWS_PER_W
    xload = pltpu.make_async_copy(
        x_hbm.at[pl.ds(row0, ROWS_PER_W)], xrows_v, sem_a
    )
    xload.start()

    ones16 = jnp.ones((L,), jnp.float32)
    zeros16 = jnp.zeros((L,), jnp.float32)
    col_iota = lax.iota(jnp.int32, L)

    def zero(buf):
        def body(i, carry):
            for j in range(C // L):
                buf[i, pl.ds(j * L, L)] = jnp.zeros((L,), jnp.float32)
            return carry

        lax.fori_loop(0, T_CHUNK, body, 0)

    def put(c, buf, val):
        r = c // N_CHUNK
        t0 = (c % N_CHUNK) * T_CHUNK
        for j in range(C // L):
            xv = xrows_v[r, pl.ds(j * L, L)]
            rr = (xv / TIME_STEP).astype(jnp.int32) - t0
            m = (rr >= 0) & (rr < T_CHUNK)
            cols = col_iota + (j * L)
            plsc.store_scatter(buf, [rr, cols], val, mask=m)

    def copy(c, buf, sem):
        r = c // N_CHUNK
        t0 = pl.multiple_of((c % N_CHUNK) * T_CHUNK, T_CHUNK)
        return pltpu.make_async_copy(
            buf, out_hbm.at[row0 + r, pl.ds(t0, T_CHUNK)], sem
        )

    zero(buf_a)
    xload.wait()
    put(0, buf_a, ones16)
    copy(0, buf_a, sem_a).start()
    zero(buf_b)
    put(1, buf_b, ones16)
    copy(1, buf_b, sem_b).start()

    def pair(p, carry):
        c0 = 2 * p
        copy(c0 - 2, buf_a, sem_a).wait()
        put(c0 - 2, buf_a, zeros16)
        put(c0, buf_a, ones16)
        copy(c0, buf_a, sem_a).start()
        copy(c0 - 1, buf_b, sem_b).wait()
        put(c0 - 1, buf_b, zeros16)
        put(c0 + 1, buf_b, ones16)
        copy(c0 + 1, buf_b, sem_b).start()
        return carry

    lax.fori_loop(1, CHUNKS // 2, pair, 0)

    copy(CHUNKS - 2, buf_a, sem_a).wait()
    copy(CHUNKS - 1, buf_b, sem_b).wait()


def kernel(x):
    return _sc_one_hot(x)
